# baseline (speedup 1.0000x reference)
import functools
import jax
import jax.numpy as jnp
from jax import lax
from jax.experimental import pallas as pl
from jax.experimental.pallas import tpu as pltpu
from jax.experimental.pallas import tpu_sc as plsc

NC, NS, L = 2, 16, 16
NW = NC * NS
BATCH = 4
SEQ = 8192
D = 1024
SPW = SEQ // NW
CH = 4
NCH = SPW // CH
NP = 4
UNROLL = 16


def _sc_add(w2d, pe2d):
    mesh = plsc.VectorSubcoreMesh(core_axis_name="c", subcore_axis_name="s",
                                  num_cores=NC, num_subcores=NS)

    @functools.partial(
        pl.kernel,
        out_type=jax.ShapeDtypeStruct((BATCH * SEQ, D), jnp.float32),
        mesh=mesh,
        scratch_types=(
            [pltpu.VMEM((BATCH * CH, D), jnp.float32) for _ in range(NP)]
            + [pltpu.VMEM((CH, D), jnp.float32) for _ in range(NP)]
            + [pltpu.SemaphoreType.DMA for _ in range(3 * NP)]
        ),
    )
    def k(w_hbm, pe_hbm, out_hbm, *scratch):
        wbufs = list(scratch[:NP])
        pbufs = list(scratch[NP:2 * NP])
        sems = list(scratch[2 * NP:])
        swl = sems[:NP]
        spl = sems[NP:2 * NP]
        sst = sems[2 * NP:]

        wid = lax.axis_index("s") * NC + lax.axis_index("c")
        s0 = wid * SPW

        def issue_loads(g, h):
            r0 = s0 + g * CH
            pltpu.async_copy(pe_hbm.at[pl.ds(r0, CH)], pbufs[h], spl[h])
            for b in range(BATCH):
                pltpu.async_copy(
                    w_hbm.at[pl.ds(b * SEQ + r0, CH)],
                    wbufs[h].at[pl.ds(b * CH, CH)], swl[h])

        def wait_loads(g, h):
            r0 = s0 + g * CH
            pltpu.make_async_copy(
                pe_hbm.at[pl.ds(r0, CH)], pbufs[h], spl[h]).wait()
            for b in range(BATCH):
                pltpu.make_async_copy(
                    w_hbm.at[pl.ds(b * SEQ + r0, CH)],
                    wbufs[h].at[pl.ds(b * CH, CH)], swl[h]).wait()

        def issue_stores(g, h):
            r0 = s0 + g * CH
            for b in range(BATCH):
                pltpu.async_copy(
                    wbufs[h].at[pl.ds(b * CH, CH)],
                    out_hbm.at[pl.ds(b * SEQ + r0, CH)], sst[h])

        def wait_stores(g, h):
            r0 = s0 + g * CH
            for b in range(BATCH):
                pltpu.make_async_copy(
                    wbufs[h].at[pl.ds(b * CH, CH)],
                    out_hbm.at[pl.ds(b * SEQ + r0, CH)], sst[h]).wait()

        def alu(h):
            wb, pb = wbufs[h], pbufs[h]
            for r in range(CH):
                @plsc.parallel_loop(0, D, step=L, unroll=UNROLL)
                def _(i):
                    pv = pb[r, pl.ds(i, L)]
                    for b in range(BATCH):
                        wb[b * CH + r, pl.ds(i, L)] = (
                            wb[b * CH + r, pl.ds(i, L)] + pv)

        def process(g, h, h2, steady):
            if steady:
                wait_stores(g - 2, h2)
            issue_loads(g + 2, h2)
            wait_loads(g, h)
            alu(h)
            issue_stores(g, h)

        issue_loads(0, 0)
        issue_loads(1, 1)
        issue_loads(2, 2)
        wait_loads(0, 0)
        alu(0)
        issue_stores(0, 0)
        issue_loads(3, 3)
        wait_loads(1, 1)
        alu(1)
        issue_stores(1, 1)

        def body(j, _):
            for kk in range(4):
                g = 4 * j + 2 + kk
                process(g, (2 + kk) % NP, kk % NP, True)
            return 0

        lax.fori_loop(0, (NCH - 4) // 4, body, 0)

        for g in (NCH - 2, NCH - 1):
            h = g % NP
            wait_stores(g - 2, (g - 2) % NP)
            wait_loads(g, h)
            alu(h)
            issue_stores(g, h)
        wait_stores(NCH - 2, (NCH - 2) % NP)
        wait_stores(NCH - 1, (NCH - 1) % NP)

    return k(w2d, pe2d)


def kernel(weights, pe):
    b, s, d = weights.shape
    out = _sc_add(weights.reshape(b * s, d), pe)
    return out.reshape(b, s, d)


# speedup vs baseline: 1.0932x; 1.0932x over previous
import functools
import jax
import jax.numpy as jnp
from jax import lax
from jax.experimental import pallas as pl
from jax.experimental.pallas import tpu as pltpu
from jax.experimental.pallas import tpu_sc as plsc

NC, NS, L = 2, 16, 16
NW = NC * NS
BATCH = 4
SEQ = 8192
D = 1024
SPW = SEQ // NW
CH = 4
NCH = SPW // CH
NP = 4
UNROLL = 8


def _sc_add(w2d, pe2d):
    mesh = plsc.VectorSubcoreMesh(core_axis_name="c", subcore_axis_name="s",
                                  num_cores=NC, num_subcores=NS)

    @functools.partial(
        pl.kernel,
        out_type=jax.ShapeDtypeStruct((BATCH * SEQ, D), jnp.float32),
        mesh=mesh,
        scratch_types=(
            [pltpu.VMEM((BATCH * CH, D), jnp.float32) for _ in range(NP)]
            + [pltpu.VMEM((CH, D), jnp.float32) for _ in range(NP)]
            + [pltpu.SemaphoreType.DMA for _ in range(3 * NP)]
        ),
    )
    def k(w_hbm, pe_hbm, out_hbm, *scratch):
        wbufs = list(scratch[:NP])
        pbufs = list(scratch[NP:2 * NP])
        sems = list(scratch[2 * NP:])
        swl = sems[:NP]
        spl = sems[NP:2 * NP]
        sst = sems[2 * NP:]

        wid = lax.axis_index("s") * NC + lax.axis_index("c")
        s0 = wid * SPW

        def issue_loads(g, h):
            r0 = s0 + g * CH
            pltpu.async_copy(pe_hbm.at[pl.ds(r0, CH)], pbufs[h], spl[h])
            for b in range(BATCH):
                pltpu.async_copy(
                    w_hbm.at[pl.ds(b * SEQ + r0, CH)],
                    wbufs[h].at[pl.ds(b * CH, CH)], swl[h])

        def wait_loads(g, h):
            r0 = s0 + g * CH
            pltpu.make_async_copy(
                pe_hbm.at[pl.ds(r0, CH)], pbufs[h], spl[h]).wait()
            for b in range(BATCH):
                pltpu.make_async_copy(
                    w_hbm.at[pl.ds(b * SEQ + r0, CH)],
                    wbufs[h].at[pl.ds(b * CH, CH)], swl[h]).wait()

        def issue_stores(g, h):
            r0 = s0 + g * CH
            for b in range(BATCH):
                pltpu.async_copy(
                    wbufs[h].at[pl.ds(b * CH, CH)],
                    out_hbm.at[pl.ds(b * SEQ + r0, CH)], sst[h])

        def wait_stores(g, h):
            r0 = s0 + g * CH
            for b in range(BATCH):
                pltpu.make_async_copy(
                    wbufs[h].at[pl.ds(b * CH, CH)],
                    out_hbm.at[pl.ds(b * SEQ + r0, CH)], sst[h]).wait()

        def alu(h):
            wb, pb = wbufs[h], pbufs[h]
            for r in range(CH):
                @plsc.parallel_loop(0, D, step=L, unroll=UNROLL)
                def _(i):
                    pv = pb[r, pl.ds(i, L)]
                    for b in range(BATCH):
                        wb[b * CH + r, pl.ds(i, L)] = (
                            wb[b * CH + r, pl.ds(i, L)] + pv)

        def process(g, h, h2, steady):
            if steady:
                wait_stores(g - 2, h2)
            issue_loads(g + 2, h2)
            wait_loads(g, h)
            alu(h)
            issue_stores(g, h)

        issue_loads(0, 0)
        issue_loads(1, 1)
        issue_loads(2, 2)
        wait_loads(0, 0)
        alu(0)
        issue_stores(0, 0)
        issue_loads(3, 3)
        wait_loads(1, 1)
        alu(1)
        issue_stores(1, 1)

        def body(j, _):
            for kk in range(4):
                g = 4 * j + 2 + kk
                process(g, (2 + kk) % NP, kk % NP, True)
            return 0

        lax.fori_loop(0, (NCH - 4) // 4, body, 0)

        for g in (NCH - 2, NCH - 1):
            h = g % NP
            wait_stores(g - 2, (g - 2) % NP)
            wait_loads(g, h)
            alu(h)
            issue_stores(g, h)
        wait_stores(NCH - 2, (NCH - 2) % NP)
        wait_stores(NCH - 1, (NCH - 1) % NP)

    return k(w2d, pe2d)


def kernel(weights, pe):
    b, s, d = weights.shape
    out = _sc_add(weights.reshape(b * s, d), pe)
    return out.reshape(b, s, d)
